# baseline (speedup 1.0000x reference)
import functools

import jax
import jax.numpy as jnp
import numpy as np
from jax import lax
from jax.experimental import pallas as pl
from jax.experimental.pallas import tpu as pltpu
from jax.experimental.pallas import tpu_sc as plsc

B, NQ, DM, M, P, HH, WW = 4, 4096, 256, 8, 4, 64, 64
D = DM // M
E = D // 2
MP = M * P
QB = 512
NPIX = HH * WW
QCHUNK = 1024

_GSUM = np.kron(np.eye(M, dtype=np.float32),
                np.ones((P, P), dtype=np.float32))

_TDIMS = (((0,), (1,)), ((), ()))
_CDIMS = (((0,), (0,)), ((), ()))


def _axis_slots(loc):
    pix = loc * 64.0 - 0.5
    t0 = jnp.floor(pix)
    f = pix - t0
    w0 = 1.0 - f
    w1 = f
    v0 = (t0 >= 0.0) & (t0 <= 63.0)
    v1 = (t0 >= -1.0) & (t0 <= 62.0)
    base = jnp.clip(t0, 0.0, 62.0)
    ws0 = (jnp.where(v0 & (base == t0), w0, 0.0)
           + jnp.where(v1 & (base == t0 + 1.0), w1, 0.0))
    ws1 = (jnp.where(v0 & (base + 1.0 == t0), w0, 0.0)
           + jnp.where(v1 & (base == t0), w1, 0.0))
    return base.astype(jnp.int32), ws0, ws1


def _bf16_bits(x):
    u = lax.bitcast_convert_type(x, jnp.uint32)
    return (u + jnp.uint32(0x7FFF) + ((u >> 16) & jnp.uint32(1))) >> 16


def _proj_body(q_ref, rp_ref, Wv_ref, bv_ref,
               Wsx_ref, bsx_ref, Wsy_ref, bsy_ref, Wa_ref, ba_ref, G_ref,
               tab_ref, k0_ref, w0_ref, w1_ref, w2_ref, w3_ref,
               locx_ref, locy_ref, aw_ref):
    q = q_ref[0]
    val = (lax.dot_general(Wv_ref[...], q, _TDIMS,
                           preferred_element_type=jnp.float32) + bv_ref[...])
    word = (_bf16_bits(val[:DM // 2, :])
            | (_bf16_bits(val[DM // 2:, :]) << 16))
    tab_ref[0] = lax.bitcast_convert_type(word, jnp.int32)

    offx = (lax.dot_general(Wsx_ref[...], q, _TDIMS,
                            preferred_element_type=jnp.float32) + bsx_ref[...])
    offy = (lax.dot_general(Wsy_ref[...], q, _TDIMS,
                            preferred_element_type=jnp.float32) + bsy_ref[...])
    rp = rp_ref[0]
    locx = rp[0:1, :] + offx * (1.0 / WW)
    locy = rp[1:2, :] + offy * (1.0 / HH)
    locx_ref[0] = locx.T
    locy_ref[0] = locy.T

    logits = (lax.dot_general(Wa_ref[...], q, _TDIMS,
                              preferred_element_type=jnp.float32) + ba_ref[...])
    logits = logits - jnp.max(logits, axis=0, keepdims=True)
    e = jnp.exp(logits)
    denom = jnp.dot(G_ref[...], e, preferred_element_type=jnp.float32)
    aw = e / denom
    aw_ref[0] = aw.T

    bx, wsx0, wsx1 = _axis_slots(locx)
    by, wsy0, wsy1 = _axis_slots(locy)
    k0_ref[0] = by * WW + bx
    w0_ref[0] = aw * wsy0 * wsx0
    w1_ref[0] = aw * wsy0 * wsx1
    w2_ref[0] = aw * wsy1 * wsx0
    w3_ref[0] = aw * wsy1 * wsx1


def _proj(query, rp_T, Wv, bv, Wsx, bsx, Wsy, bsy, Wa, ba):
    grid = (B, NQ // QB)
    tspec = pl.BlockSpec((1, MP, QB), lambda b, i: (b, 0, i))
    uspec = pl.BlockSpec((1, QB, MP), lambda b, i: (b, i, 0))
    full2 = lambda shape: pl.BlockSpec(shape, lambda b, i: (0, 0))
    out_shapes = (
        jax.ShapeDtypeStruct((B, DM // 2, NQ), jnp.int32),
        jax.ShapeDtypeStruct((B, MP, NQ), jnp.int32),
        jax.ShapeDtypeStruct((B, MP, NQ), jnp.float32),
        jax.ShapeDtypeStruct((B, MP, NQ), jnp.float32),
        jax.ShapeDtypeStruct((B, MP, NQ), jnp.float32),
        jax.ShapeDtypeStruct((B, MP, NQ), jnp.float32),
        jax.ShapeDtypeStruct((B, NQ, MP), jnp.float32),
        jax.ShapeDtypeStruct((B, NQ, MP), jnp.float32),
        jax.ShapeDtypeStruct((B, NQ, MP), jnp.float32),
    )
    return pl.pallas_call(
        _proj_body,
        grid=grid,
        in_specs=[
            pl.BlockSpec((1, QB, DM), lambda b, i: (b, i, 0)),
            pl.BlockSpec((1, 2, QB), lambda b, i: (b, 0, i)),
            full2((DM, DM)),
            full2((DM, 1)),
            full2((DM, MP)),
            full2((MP, 1)),
            full2((DM, MP)),
            full2((MP, 1)),
            full2((DM, MP)),
            full2((MP, 1)),
            full2((MP, MP)),
        ],
        out_specs=(pl.BlockSpec((1, DM // 2, QB), lambda b, i: (b, 0, i)),
                   tspec, tspec, tspec, tspec, tspec,
                   uspec, uspec, uspec),
        out_shape=out_shapes,
    )(query, rp_T, Wv, bv, Wsx, bsx, Wsy, bsy, Wa, ba, _GSUM)


def _outproj_body(x_ref, Wo_ref, bo_ref, o_ref):
    o_ref[0] = (lax.dot_general(x_ref[0], Wo_ref[...], _CDIMS,
                                preferred_element_type=jnp.float32)
                + bo_ref[...])


QB2 = 1024


def _outproj(x_T, Wo, bo):
    nb = x_T.shape[0]
    grid = (nb, NQ // QB2)
    return pl.pallas_call(
        _outproj_body,
        grid=grid,
        in_specs=[
            pl.BlockSpec((1, DM, QB2), lambda b, i: (b, 0, i)),
            pl.BlockSpec((DM, DM), lambda b, i: (0, 0)),
            pl.BlockSpec((1, DM), lambda b, i: (0, 0)),
        ],
        out_specs=pl.BlockSpec((1, QB2, DM), lambda b, i: (b, i, 0)),
        out_shape=jax.ShapeDtypeStruct((nb, NQ, DM), jnp.float32),
    )(x_T, Wo, bo)


NCHUNK = NQ // QCHUNK
EH = E // 2


def _sc_body(tab_t, k0_t, w0_t, w1_t, w2_t, w3_t, out_hbm,
             table_v, idx_v, wv_v, out_v, sem_in, sem_out):
    wid = lax.axis_index("s") * 2 + lax.axis_index("c")
    b = wid // 16
    m = (wid % 16) // 2
    h = wid % 2
    w_ts = (w0_t, w1_t, w2_t, w3_t)

    def stage_copies(chunk, buf):
        cb = chunk * QCHUNK
        cps = [pltpu.make_async_copy(
            k0_t.at[b, pl.ds(m * P, P), pl.ds(cb, QCHUNK)],
            idx_v.at[buf], sem_in)]
        for s, w_t in enumerate(w_ts):
            cps.append(pltpu.make_async_copy(
                w_t.at[b, pl.ds(m * P, P), pl.ds(cb, QCHUNK)],
                wv_v.at[buf, s], sem_in))
        return cps

    def out_copies(chunk, buf):
        cb = chunk * QCHUNK
        return [
            pltpu.make_async_copy(
                out_v.at[buf, pl.ds(0, EH)],
                out_hbm.at[b, pl.ds(m * D + h * EH, EH), pl.ds(cb, QCHUNK)],
                sem_out),
            pltpu.make_async_copy(
                out_v.at[buf, pl.ds(EH, EH)],
                out_hbm.at[b, pl.ds(m * D + E + h * EH, EH),
                           pl.ds(cb, QCHUNK)],
                sem_out),
        ]

    for cp in stage_copies(0, 0):
        cp.start()
    pltpu.sync_copy(tab_t.at[b, pl.ds(m * E + h * EH, EH), :], table_v)

    def pair_body(cp_i, carry):
        for sub in range(2):
            chunk = cp_i * 2 + sub
            buf = sub

            @pl.when(chunk + 1 < NCHUNK)
            def _():
                for cp in stage_copies(chunk + 1, 1 - buf):
                    cp.start()

            for cp in stage_copies(chunk, buf):
                cp.wait()

            @pl.when(chunk >= 2)
            def _():
                for cp in out_copies(chunk - 2, buf):
                    cp.wait()

            def g_body(g, carry2):
                gb = g * 16
                acc = [jnp.zeros((16,), jnp.float32) for _ in range(2 * EH)]
                for p in range(P):
                    kv = idx_v[buf, p, pl.ds(gb, 16)]
                    for s, off in enumerate((0, 1, WW, WW + 1)):
                        idx = kv + off
                        wv = wv_v[buf, s, p, pl.ds(gb, 16)]
                        for j in range(EH):
                            gw = plsc.load_gather(
                                table_v,
                                [jnp.full((16,), j, jnp.int32), idx])
                            lo = plsc.bitcast(lax.shift_left(gw, 16),
                                              jnp.float32)
                            hi = plsc.bitcast(gw & jnp.int32(-65536),
                                              jnp.float32)
                            acc[j] = acc[j] + wv * lo
                            acc[EH + j] = acc[EH + j] + wv * hi
                for j in range(2 * EH):
                    out_v[buf, j, pl.ds(gb, 16)] = acc[j]
                return carry2

            lax.fori_loop(0, QCHUNK // 16, g_body, 0)
            for cp in out_copies(chunk, buf):
                cp.start()
        return carry

    lax.fori_loop(0, NCHUNK // 2, pair_body, 0)
    for cp in out_copies(NCHUNK - 2, 0):
        cp.wait()
    for cp in out_copies(NCHUNK - 1, 1):
        cp.wait()


def _sc_gather(tab, k0_t, w0_t, w1_t, w2_t, w3_t):
    mesh = plsc.VectorSubcoreMesh(core_axis_name="c", subcore_axis_name="s")
    fn = functools.partial(
        pl.kernel,
        mesh=mesh,
        compiler_params=pltpu.CompilerParams(needs_layout_passes=False),
        out_type=jax.ShapeDtypeStruct((B // 2, DM, NQ), jnp.float32),
        name="scgather",
        scratch_types=[
            pltpu.VMEM((EH, NPIX), jnp.int32),
            pltpu.VMEM((2, P, QCHUNK), jnp.int32),
            pltpu.VMEM((2, 4, P, QCHUNK), jnp.float32),
            pltpu.VMEM((2, 2 * EH, QCHUNK), jnp.float32),
            pltpu.SemaphoreType.DMA,
            pltpu.SemaphoreType.DMA,
        ],
    )(_sc_body)
    return fn(tab, k0_t, w0_t, w1_t, w2_t, w3_t)


def kernel(context, context_mask, query, reference_points, Wv, bv, Ws, bs,
           Wa, ba, Wo, bo):
    rp_T = reference_points.reshape(B, NQ, 2).transpose(0, 2, 1)
    Wv4 = Wv.reshape(DM, M, 2, E)
    Wv_p = jnp.concatenate([Wv4[:, :, 0, :].reshape(DM, DM // 2),
                            Wv4[:, :, 1, :].reshape(DM, DM // 2)], axis=1)
    bv4 = bv.reshape(M, 2, E)
    bv_p = jnp.concatenate([bv4[:, 0, :].reshape(DM // 2),
                            bv4[:, 1, :].reshape(DM // 2)]).reshape(DM, 1)
    Wsx = Ws[:, 0::2]
    Wsy = Ws[:, 1::2]
    bsx = bs[0::2].reshape(MP, 1)
    bsy = bs[1::2].reshape(MP, 1)

    (tab, k0_T, w0_T, w1_T, w2_T, w3_T, locx, locy, aw) = _proj(
        query, rp_T, Wv_p, bv_p, Wsx, bsx, Wsy, bsy,
        Wa, ba.reshape(MP, 1))

    bo2 = bo.reshape(1, DM)
    outg_a = _sc_gather(tab[:2], k0_T[:2], w0_T[:2], w1_T[:2],
                        w2_T[:2], w3_T[:2])
    outg_b = _sc_gather(tab[2:], k0_T[2:], w0_T[2:], w1_T[2:],
                        w2_T[2:], w3_T[2:])
    final = jnp.concatenate([_outproj(outg_a, Wo, bo2),
                             _outproj(outg_b, Wo, bo2)], axis=0)

    sampling_locations = (jnp.stack([locx, locy], axis=-1)
                          .reshape(B, NQ, M, 1, P, 2))
    attention_weights = aw.reshape(B, NQ, M, 1, P)
    return (final, sampling_locations, attention_weights)


# speedup vs baseline: 1.0941x; 1.0941x over previous
import functools

import jax
import jax.numpy as jnp
import numpy as np
from jax import lax
from jax.experimental import pallas as pl
from jax.experimental.pallas import tpu as pltpu
from jax.experimental.pallas import tpu_sc as plsc

B, NQ, DM, M, P, HH, WW = 4, 4096, 256, 8, 4, 64, 64
D = DM // M
E = D // 2
MP = M * P
QB = 512
NPIX = HH * WW
QCHUNK = 512

_GSUM = np.kron(np.eye(M, dtype=np.float32),
                np.ones((P, P), dtype=np.float32))

_TDIMS = (((0,), (1,)), ((), ()))
_CDIMS = (((0,), (0,)), ((), ()))


def _axis_slots(loc):
    pix = loc * 64.0 - 0.5
    t0 = jnp.floor(pix)
    f = pix - t0
    w0 = 1.0 - f
    w1 = f
    v0 = (t0 >= 0.0) & (t0 <= 63.0)
    v1 = (t0 >= -1.0) & (t0 <= 62.0)
    base = jnp.clip(t0, 0.0, 62.0)
    ws0 = (jnp.where(v0 & (base == t0), w0, 0.0)
           + jnp.where(v1 & (base == t0 + 1.0), w1, 0.0))
    ws1 = (jnp.where(v0 & (base + 1.0 == t0), w0, 0.0)
           + jnp.where(v1 & (base == t0), w1, 0.0))
    return base.astype(jnp.int32), ws0, ws1


def _bf16_bits(x):
    u = lax.bitcast_convert_type(x, jnp.uint32)
    return (u + jnp.uint32(0x7FFF) + ((u >> 16) & jnp.uint32(1))) >> 16


def _proj_body(q_ref, rp_ref, Wv_ref, bv_ref,
               Wsx_ref, bsx_ref, Wsy_ref, bsy_ref, Wa_ref, ba_ref, G_ref,
               tab_ref, k0_ref, w0_ref, w1_ref, w2_ref, w3_ref,
               locx_ref, locy_ref, aw_ref):
    q = q_ref[0]
    val = (lax.dot_general(Wv_ref[...], q, _TDIMS,
                           preferred_element_type=jnp.float32) + bv_ref[...])
    word = (_bf16_bits(val[:DM // 2, :])
            | (_bf16_bits(val[DM // 2:, :]) << 16))
    tab_ref[0] = lax.bitcast_convert_type(word, jnp.int32)

    offx = (lax.dot_general(Wsx_ref[...], q, _TDIMS,
                            preferred_element_type=jnp.float32) + bsx_ref[...])
    offy = (lax.dot_general(Wsy_ref[...], q, _TDIMS,
                            preferred_element_type=jnp.float32) + bsy_ref[...])
    rp = rp_ref[0]
    locx = rp[0:1, :] + offx * (1.0 / WW)
    locy = rp[1:2, :] + offy * (1.0 / HH)
    locx_ref[0] = locx.T
    locy_ref[0] = locy.T

    logits = (lax.dot_general(Wa_ref[...], q, _TDIMS,
                              preferred_element_type=jnp.float32) + ba_ref[...])
    logits = logits - jnp.max(logits, axis=0, keepdims=True)
    e = jnp.exp(logits)
    denom = jnp.dot(G_ref[...], e, preferred_element_type=jnp.float32)
    aw = e / denom
    aw_ref[0] = aw.T

    bx, wsx0, wsx1 = _axis_slots(locx)
    by, wsy0, wsy1 = _axis_slots(locy)
    k0_ref[0] = by * WW + bx
    w0_ref[0] = aw * wsy0 * wsx0
    w1_ref[0] = aw * wsy0 * wsx1
    w2_ref[0] = aw * wsy1 * wsx0
    w3_ref[0] = aw * wsy1 * wsx1


def _proj(query, rp_T, Wv, bv, Wsx, bsx, Wsy, bsy, Wa, ba):
    grid = (B, NQ // QB)
    tspec = pl.BlockSpec((1, MP, QB), lambda b, i: (b, 0, i))
    uspec = pl.BlockSpec((1, QB, MP), lambda b, i: (b, i, 0))
    full2 = lambda shape: pl.BlockSpec(shape, lambda b, i: (0, 0))
    out_shapes = (
        jax.ShapeDtypeStruct((B, DM // 2, NQ), jnp.int32),
        jax.ShapeDtypeStruct((B, MP, NQ), jnp.int32),
        jax.ShapeDtypeStruct((B, MP, NQ), jnp.float32),
        jax.ShapeDtypeStruct((B, MP, NQ), jnp.float32),
        jax.ShapeDtypeStruct((B, MP, NQ), jnp.float32),
        jax.ShapeDtypeStruct((B, MP, NQ), jnp.float32),
        jax.ShapeDtypeStruct((B, NQ, MP), jnp.float32),
        jax.ShapeDtypeStruct((B, NQ, MP), jnp.float32),
        jax.ShapeDtypeStruct((B, NQ, MP), jnp.float32),
    )
    return pl.pallas_call(
        _proj_body,
        grid=grid,
        in_specs=[
            pl.BlockSpec((1, QB, DM), lambda b, i: (b, i, 0)),
            pl.BlockSpec((1, 2, QB), lambda b, i: (b, 0, i)),
            full2((DM, DM)),
            full2((DM, 1)),
            full2((DM, MP)),
            full2((MP, 1)),
            full2((DM, MP)),
            full2((MP, 1)),
            full2((DM, MP)),
            full2((MP, 1)),
            full2((MP, MP)),
        ],
        out_specs=(pl.BlockSpec((1, DM // 2, QB), lambda b, i: (b, 0, i)),
                   tspec, tspec, tspec, tspec, tspec,
                   uspec, uspec, uspec),
        out_shape=out_shapes,
    )(query, rp_T, Wv, bv, Wsx, bsx, Wsy, bsy, Wa, ba, _GSUM)


def _outproj_body(x_ref, Wo_ref, bo_ref, o_ref):
    o_ref[0] = (lax.dot_general(x_ref[0], Wo_ref[...], _CDIMS,
                                preferred_element_type=jnp.float32)
                + bo_ref[...])


QB2 = 1024


def _outproj(x_T, Wo, bo):
    nb = x_T.shape[0]
    grid = (nb, NQ // QB2)
    return pl.pallas_call(
        _outproj_body,
        grid=grid,
        in_specs=[
            pl.BlockSpec((1, DM, QB2), lambda b, i: (b, 0, i)),
            pl.BlockSpec((DM, DM), lambda b, i: (0, 0)),
            pl.BlockSpec((1, DM), lambda b, i: (0, 0)),
        ],
        out_specs=pl.BlockSpec((1, QB2, DM), lambda b, i: (b, i, 0)),
        out_shape=jax.ShapeDtypeStruct((nb, NQ, DM), jnp.float32),
    )(x_T, Wo, bo)


NCHUNK = NQ // QCHUNK


def _sc_body(tab_t, k0_t, w0_t, w1_t, w2_t, w3_t, out_hbm,
             table_v, idx_v, wv_v, out_v, sem_in, sem_out):
    wid = lax.axis_index("s") * 2 + lax.axis_index("c")
    b = wid // M
    m = wid % M
    w_ts = (w0_t, w1_t, w2_t, w3_t)

    def stage_copies(chunk, buf):
        cb = chunk * QCHUNK
        cps = [pltpu.make_async_copy(
            k0_t.at[b, pl.ds(m * P, P), pl.ds(cb, QCHUNK)],
            idx_v.at[buf], sem_in)]
        for s, w_t in enumerate(w_ts):
            cps.append(pltpu.make_async_copy(
                w_t.at[b, pl.ds(m * P, P), pl.ds(cb, QCHUNK)],
                wv_v.at[buf, s], sem_in))
        return cps

    def out_copy(chunk, buf):
        return pltpu.make_async_copy(
            out_v.at[buf],
            out_hbm.at[b, pl.ds(m * D, D), pl.ds(chunk * QCHUNK, QCHUNK)],
            sem_out)

    for cp in stage_copies(0, 0):
        cp.start()
    pltpu.sync_copy(tab_t.at[b, pl.ds(m * E, E), :], table_v)

    def pair_body(cp_i, carry):
        for sub in range(2):
            chunk = cp_i * 2 + sub
            buf = sub

            @pl.when(chunk + 1 < NCHUNK)
            def _():
                for cp in stage_copies(chunk + 1, 1 - buf):
                    cp.start()

            for cp in stage_copies(chunk, buf):
                cp.wait()

            @pl.when(chunk >= 2)
            def _():
                out_copy(chunk - 2, buf).wait()

            def g_body(g, carry2):
                gb = g * 16
                for phase in range(2):
                    e0 = phase * (E // 2)
                    acc = [jnp.zeros((16,), jnp.float32) for _ in range(E)]
                    for p in range(P):
                        kv = idx_v[buf, p, pl.ds(gb, 16)]
                        for s, off in enumerate((0, 1, WW, WW + 1)):
                            idx = kv + off
                            wv = wv_v[buf, s, p, pl.ds(gb, 16)]
                            for j in range(E // 2):
                                gw = plsc.load_gather(
                                    table_v,
                                    [jnp.full((16,), e0 + j, jnp.int32), idx])
                                lo = plsc.bitcast(lax.shift_left(gw, 16),
                                                  jnp.float32)
                                hi = plsc.bitcast(gw & jnp.int32(-65536),
                                                  jnp.float32)
                                acc[2 * j] = acc[2 * j] + wv * lo
                                acc[2 * j + 1] = acc[2 * j + 1] + wv * hi
                    for j in range(E // 2):
                        out_v[buf, e0 + j, pl.ds(gb, 16)] = acc[2 * j]
                        out_v[buf, E + e0 + j, pl.ds(gb, 16)] = acc[2 * j + 1]
                return carry2

            lax.fori_loop(0, QCHUNK // 16, g_body, 0)
            out_copy(chunk, buf).start()
        return carry

    lax.fori_loop(0, NCHUNK // 2, pair_body, 0)
    out_copy(NCHUNK - 2, 0).wait()
    out_copy(NCHUNK - 1, 1).wait()


def _sc_gather(tab, k0_t, w0_t, w1_t, w2_t, w3_t):
    mesh = plsc.VectorSubcoreMesh(core_axis_name="c", subcore_axis_name="s")
    fn = functools.partial(
        pl.kernel,
        mesh=mesh,
        compiler_params=pltpu.CompilerParams(needs_layout_passes=False),
        out_type=jax.ShapeDtypeStruct((B, DM, NQ), jnp.float32),
        name="scgather",
        scratch_types=[
            pltpu.VMEM((E, NPIX), jnp.int32),
            pltpu.VMEM((2, P, QCHUNK), jnp.int32),
            pltpu.VMEM((2, 4, P, QCHUNK), jnp.float32),
            pltpu.VMEM((2, D, QCHUNK), jnp.float32),
            pltpu.SemaphoreType.DMA,
            pltpu.SemaphoreType.DMA,
        ],
    )(_sc_body)
    return fn(tab, k0_t, w0_t, w1_t, w2_t, w3_t)


def kernel(context, context_mask, query, reference_points, Wv, bv, Ws, bs,
           Wa, ba, Wo, bo):
    rp_T = reference_points.reshape(B, NQ, 2).transpose(0, 2, 1)
    Wv4 = Wv.reshape(DM, M, 2, E)
    Wv_p = jnp.concatenate([Wv4[:, :, 0, :].reshape(DM, DM // 2),
                            Wv4[:, :, 1, :].reshape(DM, DM // 2)], axis=1)
    bv4 = bv.reshape(M, 2, E)
    bv_p = jnp.concatenate([bv4[:, 0, :].reshape(DM // 2),
                            bv4[:, 1, :].reshape(DM // 2)]).reshape(DM, 1)
    Wsx = Ws[:, 0::2]
    Wsy = Ws[:, 1::2]
    bsx = bs[0::2].reshape(MP, 1)
    bsy = bs[1::2].reshape(MP, 1)

    (tab, k0_T, w0_T, w1_T, w2_T, w3_T, locx, locy, aw) = _proj(
        query, rp_T, Wv_p, bv_p, Wsx, bsx, Wsy, bsy,
        Wa, ba.reshape(MP, 1))

    outg_T = _sc_gather(tab, k0_T, w0_T, w1_T, w2_T, w3_T)

    final = _outproj(outg_T, Wo, bo.reshape(1, DM))

    sampling_locations = (jnp.stack([locx, locy], axis=-1)
                          .reshape(B, NQ, M, 1, P, 2))
    attention_weights = aw.reshape(B, NQ, M, 1, P)
    return (final, sampling_locations, attention_weights)
